# baseline (speedup 1.0000x reference)
import functools
import math

import jax
import jax.numpy as jnp
from jax import lax
from jax.experimental import pallas as pl
from jax.experimental.pallas import tpu as pltpu
from jax.experimental.pallas import tpu_sc as plsc

_EMBEDDING_DIM = 128
_PADDING_IDX = 0


def _sinusoid_table(num_embeddings, embedding_dim, padding_idx=None):
    half_dim = embedding_dim // 2
    scale = math.log(10000.0) / (half_dim - 1)
    inv_freq = jnp.exp(jnp.arange(half_dim, dtype=jnp.float32) * -scale)
    start = -num_embeddings // 2
    stop = num_embeddings // 2
    pos = jnp.arange(start, stop, dtype=jnp.float32)
    emb = pos[:, None] * inv_freq[None, :]
    emb = jnp.reshape(
        jnp.concatenate([jnp.sin(emb), jnp.cos(emb)], axis=1),
        (num_embeddings, -1))
    if embedding_dim % 2 == 1:
        emb = jnp.concatenate(
            [emb, jnp.zeros((num_embeddings, 1), dtype=jnp.float32)], axis=1)
    if padding_idx is not None:
        emb = emb.at[padding_idx].set(0.0)
    return emb


@functools.cache
def _row_gather_call(num_rows, start, dim):
    info = plsc.get_sparse_core_info()
    nc, ns = info.num_cores, info.num_subcores
    nw = nc * ns
    assert num_rows % nw == 0
    elems_per_w = (num_rows // nw) * dim
    mesh = plsc.VectorSubcoreMesh(core_axis_name="c", subcore_axis_name="s")

    @functools.partial(
        pl.kernel,
        mesh=mesh,
        out_type=jax.ShapeDtypeStruct((num_rows * dim,), jnp.float32),
    )
    def k(w_hbm, out_hbm):
        wid = lax.axis_index("s") * nc + lax.axis_index("c")
        base = wid * elems_per_w
        pltpu.sync_copy(w_hbm.at[pl.ds(start * dim + base, elems_per_w)],
                        out_hbm.at[pl.ds(base, elems_per_w)])

    return k


def kernel(inputs, weight):
    seq_len = inputs.shape[1]
    max_pos = _PADDING_IDX + seq_len
    origin_shift = weight.shape[0] // 2 + 1
    if max_pos > origin_shift:
        weight = _sinusoid_table(max_pos * 2, _EMBEDDING_DIM, _PADDING_IDX)
        origin_shift = weight.shape[0] // 2
    start = origin_shift - seq_len
    dim = weight.shape[1]
    flat = _row_gather_call(2 * seq_len, start, dim)(weight.reshape(-1))
    return flat.reshape(2 * seq_len, dim)


# speedup vs baseline: 3.9560x; 3.9560x over previous
import functools
import math

import jax
import jax.numpy as jnp
from jax import lax
from jax.experimental import pallas as pl
from jax.experimental.pallas import tpu as pltpu
from jax.experimental.pallas import tpu_sc as plsc

_EMBEDDING_DIM = 128
_PADDING_IDX = 0


def _sinusoid_table(num_embeddings, embedding_dim, padding_idx=None):
    half_dim = embedding_dim // 2
    scale = math.log(10000.0) / (half_dim - 1)
    inv_freq = jnp.exp(jnp.arange(half_dim, dtype=jnp.float32) * -scale)
    start = -num_embeddings // 2
    stop = num_embeddings // 2
    pos = jnp.arange(start, stop, dtype=jnp.float32)
    emb = pos[:, None] * inv_freq[None, :]
    emb = jnp.reshape(
        jnp.concatenate([jnp.sin(emb), jnp.cos(emb)], axis=1),
        (num_embeddings, -1))
    if embedding_dim % 2 == 1:
        emb = jnp.concatenate(
            [emb, jnp.zeros((num_embeddings, 1), dtype=jnp.float32)], axis=1)
    if padding_idx is not None:
        emb = emb.at[padding_idx].set(0.0)
    return emb


@functools.cache
def _row_gather_call(num_rows, start, dim):
    info = plsc.get_sparse_core_info()
    nc, ns = info.num_cores, info.num_subcores
    nw = nc * ns
    assert num_rows % nw == 0
    elems_per_w = (num_rows // nw) * dim
    mesh = plsc.VectorSubcoreMesh(core_axis_name="c", subcore_axis_name="s")

    half = elems_per_w // 2

    @functools.partial(
        pl.kernel,
        mesh=mesh,
        out_type=jax.ShapeDtypeStruct((num_rows * dim,), jnp.float32),
        scratch_types=[
            pltpu.VMEM((half,), jnp.float32),
            pltpu.VMEM((half,), jnp.float32),
            pltpu.SemaphoreType.DMA,
            pltpu.SemaphoreType.DMA,
            pltpu.SemaphoreType.DMA,
            pltpu.SemaphoreType.DMA,
        ],
    )
    def k(w_hbm, out_hbm, buf0, buf1, si0, si1, so0, so1):
        wid = lax.axis_index("s") * nc + lax.axis_index("c")
        base = wid * elems_per_w
        src = start * dim + base
        in0 = pltpu.async_copy(w_hbm.at[pl.ds(src, half)], buf0, si0)
        in1 = pltpu.async_copy(w_hbm.at[pl.ds(src + half, half)], buf1, si1)
        in0.wait()
        out0 = pltpu.async_copy(buf0, out_hbm.at[pl.ds(base, half)], so0)
        in1.wait()
        out1 = pltpu.async_copy(buf1, out_hbm.at[pl.ds(base + half, half)], so1)
        out0.wait()
        out1.wait()

    return k


def kernel(inputs, weight):
    seq_len = inputs.shape[1]
    max_pos = _PADDING_IDX + seq_len
    origin_shift = weight.shape[0] // 2 + 1
    if max_pos > origin_shift:
        weight = _sinusoid_table(max_pos * 2, _EMBEDDING_DIM, _PADDING_IDX)
        origin_shift = weight.shape[0] // 2
    start = origin_shift - seq_len
    dim = weight.shape[1]
    flat = _row_gather_call(2 * seq_len, start, dim)(weight.reshape(-1))
    return flat.reshape(2 * seq_len, dim)
